# baseline (speedup 1.0000x reference)
import jax
import jax.numpy as jnp
from jax import lax
from jax.experimental import pallas as pl
from jax.experimental.pallas import tpu as pltpu
from jax.experimental.pallas import tpu_sc as plsc

B = 256
F = 3 * 224 * 224
NCH = 98
CHK = F // NCH
NW = 32
G = 16
CPW = NCH // 2


def _mixup_body(xv, permf, l0, l1, outx,
                perm_v, l0v, l1v, idx0_v, idx1_v, oidx_v,
                a_v, b_v, o_v, sem, osem):
    wid = lax.axis_index("s") * 2 + lax.axis_index("c")
    g = wid // 2
    h = wid % 2
    pltpu.sync_copy(permf, perm_v)
    pltpu.sync_copy(l0, l0v)
    pltpu.sync_copy(l1, l1v)
    l0r = l0v[...]
    l1r = l1v[...]
    ci = lax.iota(jnp.int32, 16)

    pv0 = perm_v[pl.ds(g * G, 16)]
    pv1 = perm_v[pl.ds(B + g * G, 16)]
    rows = g * G + ci

    def chunk_body(c, carry):
        idx0_v[...] = pv0 * NCH + c
        idx1_v[...] = pv1 * NCH + c
        oidx_v[...] = rows * NCH + c
        cp_a = pltpu.async_copy(xv.at[idx0_v], a_v, sem)
        cp_b = pltpu.async_copy(xv.at[idx1_v], b_v, sem)
        cp_a.wait()
        cp_b.wait()
        for r in range(G):
            def vec_body(k, carry2):
                s = pl.ds(k * 16, 16)
                o_v[r, s] = a_v[r, s] * l0r + b_v[r, s] * l1r
                return carry2
            lax.fori_loop(0, CHK // 16, vec_body, 0)
        pltpu.async_copy(o_v, outx.at[oidx_v], osem).wait()
        return carry

    lax.fori_loop(h * CPW, (h + 1) * CPW, chunk_body, 0)


def _labels_body(y_ref, perm_ref, y0_ref, y1_ref):
    def body(i, carry):
        y0_ref[i] = y_ref[perm_ref[0, i]]
        y1_ref[i] = y_ref[perm_ref[1, i]]
        return carry

    lax.fori_loop(0, B, body, 0)


def kernel(x, y, perm, lmb):
    xv = x.reshape(B * NCH, CHK)
    permf = perm.reshape(2 * B)
    l0 = jnp.full((16,), lmb[0], jnp.float32)
    l1 = jnp.full((16,), lmb[1], jnp.float32)
    mesh = plsc.VectorSubcoreMesh(core_axis_name="c", subcore_axis_name="s")
    f = pl.kernel(
        _mixup_body,
        mesh=mesh,
        out_type=[
            jax.ShapeDtypeStruct((B * NCH, CHK), jnp.float32),
        ],
        scratch_types=[
            pltpu.VMEM((2 * B,), jnp.int32),
            pltpu.VMEM((16,), jnp.float32),
            pltpu.VMEM((16,), jnp.float32),
            pltpu.VMEM((16,), jnp.int32),
            pltpu.VMEM((16,), jnp.int32),
            pltpu.VMEM((16,), jnp.int32),
            pltpu.VMEM((G, CHK), jnp.float32),
            pltpu.VMEM((G, CHK), jnp.float32),
            pltpu.VMEM((G, CHK), jnp.float32),
            pltpu.SemaphoreType.DMA,
            pltpu.SemaphoreType.DMA,
        ],
    )
    (outx,) = f(xv, permf, l0, l1)
    y0, y1 = pl.pallas_call(
        _labels_body,
        in_specs=[
            pl.BlockSpec(memory_space=pltpu.SMEM),
            pl.BlockSpec(memory_space=pltpu.SMEM),
        ],
        out_specs=[
            pl.BlockSpec(memory_space=pltpu.SMEM),
            pl.BlockSpec(memory_space=pltpu.SMEM),
        ],
        out_shape=[
            jax.ShapeDtypeStruct((B,), jnp.int32),
            jax.ShapeDtypeStruct((B,), jnp.int32),
        ],
    )(y, perm)
    return (outx.reshape(B, 3, 224, 224), y0, y1, lmb)


# speedup vs baseline: 2.5914x; 2.5914x over previous
import jax
import jax.numpy as jnp
from jax import lax
from jax.experimental import pallas as pl
from jax.experimental.pallas import tpu as pltpu
from jax.experimental.pallas import tpu_sc as plsc

B = 256
C = 3
H = 224
W = 224
NW = 32
RPW = B // NW
SB = 56
NT = H // SB
STEPS = RPW * C * NT


def _mixup_body(x3, permf, l0, l1, outx,
                perm_v, l0v, l1v, a0, a1, b0, b1, o0, o1,
                gsem0, gsem1, ssem0, ssem1):
    wid = lax.axis_index("s") * 2 + lax.axis_index("c")
    base = wid * RPW
    pltpu.sync_copy(permf, perm_v.at[pl.ds(0, 2 * B)])
    pltpu.sync_copy(l0, l0v)
    pltpu.sync_copy(l1, l1v)
    l0r = l0v[...]
    l1r = l1v[...]

    a_bufs = (a0, a1)
    b_bufs = (b0, b1)
    o_bufs = (o0, o1)
    gsems = (gsem0, gsem1)
    ssems = (ssem0, ssem1)

    def issue_gather(st, p):
        i = st // (C * NT)
        c = (st // NT) % C
        t = st % NT
        r0 = perm_v[pl.ds(base + i, 16)][0] * C + c
        r1 = perm_v[pl.ds(B + base + i, 16)][0] * C + c
        sl = pl.ds(t * SB, SB)
        pltpu.async_copy(x3.at[r0, sl], a_bufs[p], gsems[p])
        pltpu.async_copy(x3.at[r1, sl], b_bufs[p], gsems[p])

    def wait_gather(p):
        pltpu.make_async_copy(x3.at[0, pl.ds(0, SB)], a_bufs[p], gsems[p]).wait()
        pltpu.make_async_copy(x3.at[0, pl.ds(0, SB)], b_bufs[p], gsems[p]).wait()

    def issue_scatter(st, p):
        i = st // (C * NT)
        c = (st // NT) % C
        t = st % NT
        ro = (base + i) * C + c
        pltpu.async_copy(o_bufs[p], outx.at[ro, pl.ds(t * SB, SB)], ssems[p])

    def wait_scatter(p):
        pltpu.make_async_copy(o_bufs[p], outx.at[0, pl.ds(0, SB)], ssems[p]).wait()

    def compute(p):
        av, bv, ov = a_bufs[p], b_bufs[p], o_bufs[p]

        def row_body(r, carry):
            for k in range(W // 16):
                s = pl.ds(k * 16, 16)
                ov[r, s] = av[r, s] * l0r + bv[r, s] * l1r
            return carry

        lax.fori_loop(0, SB, row_body, 0)

    issue_gather(0, 0)

    def outer(s2, carry):
        for p in range(2):
            st = s2 * 2 + p

            @pl.when(st + 1 < STEPS)
            def _():
                issue_gather(st + 1, 1 - p)

            wait_gather(p)

            @pl.when(st >= 2)
            def _():
                wait_scatter(p)

            compute(p)
            issue_scatter(st, p)
        return carry

    lax.fori_loop(0, STEPS // 2, outer, 0)
    wait_scatter(0)
    wait_scatter(1)


def _labels_body(y_ref, perm_ref, y0_ref, y1_ref):
    def body(i, carry):
        y0_ref[i] = y_ref[perm_ref[0, i]]
        y1_ref[i] = y_ref[perm_ref[1, i]]
        return carry

    lax.fori_loop(0, B, body, 0)


def kernel(x, y, perm, lmb):
    x3 = x.reshape(B * C, H, W)
    permf = perm.reshape(2 * B)
    l0 = jnp.full((16,), lmb[0], jnp.float32)
    l1 = jnp.full((16,), lmb[1], jnp.float32)
    mesh = plsc.VectorSubcoreMesh(core_axis_name="c", subcore_axis_name="s")
    f = pl.kernel(
        _mixup_body,
        mesh=mesh,
        out_type=[
            jax.ShapeDtypeStruct((B * C, H, W), jnp.float32),
        ],
        scratch_types=[
            pltpu.VMEM((2 * B + 16,), jnp.int32),
            pltpu.VMEM((16,), jnp.float32),
            pltpu.VMEM((16,), jnp.float32),
            pltpu.VMEM((SB, W), jnp.float32),
            pltpu.VMEM((SB, W), jnp.float32),
            pltpu.VMEM((SB, W), jnp.float32),
            pltpu.VMEM((SB, W), jnp.float32),
            pltpu.VMEM((SB, W), jnp.float32),
            pltpu.VMEM((SB, W), jnp.float32),
            pltpu.SemaphoreType.DMA,
            pltpu.SemaphoreType.DMA,
            pltpu.SemaphoreType.DMA,
            pltpu.SemaphoreType.DMA,
        ],
    )
    (outx,) = f(x3, permf, l0, l1)
    y0, y1 = pl.pallas_call(
        _labels_body,
        in_specs=[
            pl.BlockSpec(memory_space=pltpu.SMEM),
            pl.BlockSpec(memory_space=pltpu.SMEM),
        ],
        out_specs=[
            pl.BlockSpec(memory_space=pltpu.SMEM),
            pl.BlockSpec(memory_space=pltpu.SMEM),
        ],
        out_shape=[
            jax.ShapeDtypeStruct((B,), jnp.int32),
            jax.ShapeDtypeStruct((B,), jnp.int32),
        ],
    )(y, perm)
    return (outx.reshape(B, C, H, W), y0, y1, lmb)
